# baseline (speedup 1.0000x reference)
import functools

import jax
import jax.numpy as jnp
from jax import lax
from jax.experimental import pallas as pl
from jax.experimental.pallas import tpu as pltpu
from jax.experimental.pallas import tpu_sc as plsc

EMB_DIM = 32
NUM_CORES = 2
NUM_SUBCORES = 16
NUM_WORKERS = NUM_CORES * NUM_SUBCORES
BLK = 128
BLKS_PER_CHUNK = 5
CHUNK = BLK * BLKS_PER_CHUNK


@functools.lru_cache(maxsize=None)
def _make_gather(B, D):
    tok_per_w = B // NUM_WORKERS
    n = tok_per_w // CHUNK
    blocks_per_w = tok_per_w // BLK
    dsub = D // 8
    mesh = plsc.VectorSubcoreMesh(core_axis_name="c", subcore_axis_name="s")

    @functools.partial(
        pl.kernel,
        out_type=jax.ShapeDtypeStruct((B * D,), jnp.float32),
        mesh=mesh,
        scratch_types=[
            pltpu.VMEM((CHUNK,), jnp.int32),
            pltpu.VMEM((CHUNK,), jnp.int32),
            pltpu.VMEM((CHUNK, D), jnp.float32),
            pltpu.VMEM((CHUNK, D), jnp.float32),
            pltpu.VMEM((CHUNK * D,), jnp.float32),
            pltpu.VMEM((CHUNK * D,), jnp.float32),
            pltpu.SemaphoreType.DMA,
            pltpu.SemaphoreType.DMA,
            pltpu.SemaphoreType.DMA,
            pltpu.SemaphoreType.DMA,
            pltpu.SemaphoreType.DMA,
            pltpu.SemaphoreType.DMA,
        ],
        compiler_params=pltpu.CompilerParams(
            use_tc_tiling_on_sc=False, needs_layout_passes=False),
    )
    def grab(ids_hbm, table_hbm, out_hbm, idx0, idx1, rows0, rows1,
             tb0, tb1, si0, si1, sg0, sg1, ss0, ss1):
        wid = lax.axis_index("s") * NUM_CORES + lax.axis_index("c")
        tok_base = wid * tok_per_w
        blk_base = wid * blocks_per_w
        idx = (idx0, idx1)
        rows = (rows0, rows1)
        tbuf = (tb0, tb1)
        sem_i = (si0, si1)
        sem_g = (sg0, sg1)
        sem_s = (ss0, ss1)
        iota16 = lax.iota(jnp.int32, 16)

        def idx_start(g, e):
            off = tok_base + g * CHUNK
            return pltpu.async_copy(
                ids_hbm.at[pl.ds(off, CHUNK)], idx[e], sem_i[e])

        def idx_drain(e):
            pltpu.make_async_copy(
                ids_hbm.at[pl.ds(0, CHUNK)], idx[e], sem_i[e]).wait()

        def gather_start(e):
            return pltpu.async_copy(
                table_hbm.at[idx[e]], rows[e], sem_g[e])

        def gather_drain(e):
            pltpu.make_async_copy(
                table_hbm.at[pl.ds(0, CHUNK)], rows[e], sem_g[e]).wait()

        def stores_drain(e):
            pltpu.make_async_copy(
                out_hbm.at[pl.ds(0, CHUNK * D)], tbuf[e], sem_s[e]).wait()

        def transpose_chunk(e):
            src = rows[e]
            dst = tbuf[e]

            def body(i, carry):
                sub = src.at[pl.ds(i * 16, 16), :]
                dst_base = (i // 8) * (D * BLK) + (i % 8) * 16
                for d in range(D):
                    val = plsc.load_gather(
                        sub, [iota16, jnp.full((16,), d, jnp.int32)])
                    off = dst_base + (d // 8) * (8 * BLK) + (d % 8) * BLK
                    dst[pl.ds(off, 16)] = val
                return carry

            lax.fori_loop(0, (CHUNK // 16), body, 0)

        def store_start(g, e):
            for blk in range(BLKS_PER_CHUNK):
                j = blk_base + g * BLKS_PER_CHUNK + blk
                tw = j // 8
                bt = j % 8
                for dt in range(dsub):
                    dst_off = ((tw * dsub + dt) * 8 + bt) * (8 * BLK)
                    src_off = blk * (D * BLK) + dt * (8 * BLK)
                    pltpu.async_copy(
                        tbuf[e].at[pl.ds(src_off, 8 * BLK)],
                        out_hbm.at[pl.ds(dst_off, 8 * BLK)],
                        sem_s[e])

        def chunkstep(g, e, drain_st, issue_idx, issue_gather):
            gather_drain(e)
            if issue_idx:
                idx_start(g + 2, e)
            if drain_st:
                stores_drain(e)
            if issue_gather:
                idx_drain(1 - e)
                gather_start(1 - e)
            transpose_chunk(e)
            store_start(g, e)

        idx_start(0, 0)
        idx_start(1, 1)
        idx_drain(0)
        gather_start(0)
        chunkstep(0, 0, drain_st=False, issue_idx=True, issue_gather=True)
        chunkstep(1, 1, drain_st=False, issue_idx=True, issue_gather=True)

        def pair(k, carry):
            g = 2 * k
            chunkstep(g, 0, drain_st=True, issue_idx=True, issue_gather=True)
            chunkstep(g + 1, 1, drain_st=True, issue_idx=True,
                      issue_gather=True)
            return carry

        lax.fori_loop(1, n // 2 - 1, pair, 0)

        chunkstep(n - 2, 0, drain_st=True, issue_idx=False, issue_gather=True)
        chunkstep(n - 1, 1, drain_st=True, issue_idx=False,
                  issue_gather=False)
        stores_drain(0)
        stores_drain(1)

    return grab


def kernel(token_ids, emb_weight):
    b, t, w = token_ids.shape
    B = token_ids.size
    D = emb_weight.shape[1]
    ids = jnp.transpose(token_ids, (1, 2, 0)).reshape(B)
    flat = _make_gather(B, D)(ids, emb_weight)
    P = flat.reshape(t * w, D // 8, b // 128, 8, 128)
    Q = jnp.transpose(P, (2, 4, 0, 1, 3))
    return Q.reshape(b, t, w, D)
